# Initial kernel scaffold; baseline (speedup 1.0000x reference)
import jax
import jax.numpy as jnp
from jax.experimental import pallas as pl


def kernel(x, edge_index, Wl, Wr, bl):
    raise NotImplementedError("write your pallas kernel here")


# speedup vs baseline: 4.6785x; 4.6785x over previous
import functools

import jax
import jax.numpy as jnp
from jax import lax
from jax.experimental import pallas as pl
from jax.experimental.pallas import tpu as pltpu
from jax.experimental.pallas import tpu_sc as plsc

N_NODES = 10000
N_PAD = 10240
D = 128
N_EDGES = 320000
NC = 2
NS = 16
NW = NC * NS
EDGES_PER_TILE = N_EDGES // NW
CHUNK = 80
STEPS = EDGES_PER_TILE // CHUNK
ROWS_PER_TILE = N_PAD // NS
ZB = 64


def _sc_aggregate(xp, src, dst, zeros, ones):
    mesh = plsc.VectorSubcoreMesh(core_axis_name="c", subcore_axis_name="s")

    @functools.partial(
        pl.kernel,
        out_type=(
            jax.ShapeDtypeStruct((NC * N_PAD, D), jnp.float32),
            jax.ShapeDtypeStruct((NC * N_PAD, D), jnp.float32),
        ),
        mesh=mesh,
        scratch_types=[
            pltpu.VMEM((CHUNK,), jnp.int32),
            pltpu.VMEM((CHUNK,), jnp.int32),
            pltpu.VMEM((CHUNK, D), jnp.float32),
            pltpu.VMEM((ZB, D), jnp.float32),
            pltpu.VMEM((ZB,), jnp.int32),
            pltpu.VMEM_SHARED((N_PAD, D), jnp.float32),
            pltpu.SemaphoreType.DMA,
        ],
    )
    def agg(x_hbm, src_hbm, dst_hbm, zero_hbm, ones_hbm, sum_out, cnt_out,
            srcv, dstv, rows, zbuf, idxv, acc_sh, sem):
        c = lax.axis_index("c")
        s = lax.axis_index("s")
        row0 = s * ROWS_PER_TILE
        tile_base = (c * NS + s) * EDGES_PER_TILE

        def fill_idx(base):
            for m in range(ZB // 16):
                idxv[pl.ds(16 * m, 16)] = lax.iota(jnp.int32, 16) + (
                    base + 16 * m)

        def zero_acc():
            @pl.loop(0, ROWS_PER_TILE // ZB)
            def zloop(k):
                fill_idx(row0 + k * ZB)
                pltpu.sync_copy(zbuf, acc_sh.at[idxv])

        def write_acc(out_ref):
            @pl.loop(0, ROWS_PER_TILE // ZB)
            def wloop(k):
                base = row0 + k * ZB
                fill_idx(base)
                pltpu.sync_copy(acc_sh.at[idxv], zbuf)
                pltpu.sync_copy(zbuf, out_ref.at[pl.ds(c * N_PAD + base, ZB)])

        pltpu.sync_copy(zero_hbm, zbuf)
        zero_acc()
        plsc.subcore_barrier()

        @pl.loop(0, STEPS)
        def step(j):
            base = tile_base + j * CHUNK
            pltpu.sync_copy(src_hbm.at[pl.ds(base, CHUNK)], srcv)
            pltpu.sync_copy(dst_hbm.at[pl.ds(base, CHUNK)], dstv)
            pltpu.async_copy(x_hbm.at[srcv], rows, sem).wait()
            pltpu.sync_copy(rows, acc_sh.at[dstv], add=True)

        plsc.subcore_barrier()
        write_acc(sum_out)
        plsc.subcore_barrier()

        pltpu.sync_copy(zero_hbm, zbuf)
        zero_acc()
        pltpu.sync_copy(ones_hbm, rows)
        plsc.subcore_barrier()

        @pl.loop(0, STEPS)
        def cstep(j):
            base = tile_base + j * CHUNK
            pltpu.sync_copy(dst_hbm.at[pl.ds(base, CHUNK)], dstv)
            pltpu.sync_copy(rows, acc_sh.at[dstv], add=True)

        plsc.subcore_barrier()
        write_acc(cnt_out)

    return agg(xp, src, dst, zeros, ones)


def _tc_kernel(sum_ref, cnt_ref, x_ref, wl_ref, wr_ref, bl_ref, out_ref):
    tot = sum_ref[0] + sum_ref[1]
    cnt = cnt_ref[0, :, 0:1] + cnt_ref[1, :, 0:1]
    recip = 1.0 / jnp.maximum(cnt, 1.0)
    mean = tot * recip
    xb = x_ref[...]
    for i in range(2):
        y = (lax.dot_general(mean, wl_ref[i], (((1,), (1,)), ((), ())),
                             preferred_element_type=jnp.float32)
             + bl_ref[i][None, :]
             + lax.dot_general(xb, wr_ref[i], (((1,), (1,)), ((), ())),
                               preferred_element_type=jnp.float32))
        out_ref[:, i * D:(i + 1) * D] = jnp.where(y >= 0.0, y, 0.01 * y)


def _tc_combine(sums, cnts, xp, Wl, Wr, bl):
    blk = 1024
    grid = N_PAD // blk
    return pl.pallas_call(
        _tc_kernel,
        grid=(grid,),
        in_specs=[
            pl.BlockSpec((NC, blk, D), lambda i: (0, i, 0)),
            pl.BlockSpec((NC, blk, D), lambda i: (0, i, 0)),
            pl.BlockSpec((blk, D), lambda i: (i, 0)),
            pl.BlockSpec((NC, D, D), lambda i: (0, 0, 0)),
            pl.BlockSpec((NC, D, D), lambda i: (0, 0, 0)),
            pl.BlockSpec((NC, D), lambda i: (0, 0)),
        ],
        out_specs=pl.BlockSpec((blk, 2 * D), lambda i: (i, 0)),
        out_shape=jax.ShapeDtypeStruct((N_PAD, 2 * D), jnp.float32),
    )(sums, cnts, xp, Wl, Wr, bl)


@jax.jit
def kernel(x, edge_index, Wl, Wr, bl):
    src = edge_index[0].astype(jnp.int32)
    dst = edge_index[1].astype(jnp.int32)
    xp = jnp.pad(x, ((0, N_PAD - N_NODES), (0, 0)))
    zeros = jnp.zeros((ZB, D), jnp.float32)
    ones = jnp.ones((CHUNK, D), jnp.float32)
    sums, cnts = _sc_aggregate(xp, src, dst, zeros, ones)
    sums = sums.reshape(NC, N_PAD, D)
    cnts = cnts.reshape(NC, N_PAD, D)
    out = _tc_combine(sums, cnts, xp, Wl, Wr, bl)
    return out[:N_NODES]
